# baseline (device time: 42504 ns/iter reference)
import jax
import jax.numpy as jnp
from jax import lax
from jax.experimental import pallas as pl
from jax.experimental.pallas import tpu as pltpu

N_DEV = 4


def kernel(x, Win0, Wout0, Win1, Wout1, Win2, Wout2):
    M, D = x.shape
    B = N_DEV * M

    def body(x_ref, win0_ref, wout0_ref, win1_ref, wout1_ref, win2_ref,
             wout2_ref, out_ref, X0, X1, X2, Pbuf, rbufs, sbufs,
             send_sems, recv_sems):
        my = lax.axis_index("i")

        barrier_sem = pltpu.get_barrier_semaphore()
        for k in range(1, N_DEV):
            pl.semaphore_signal(
                barrier_sem, inc=1,
                device_id=((my + k) % N_DEV,),
                device_id_type=pl.DeviceIdType.MESH,
            )
        pl.semaphore_wait(barrier_sem, N_DEV - 1)

        def exchange(round_idx, src_at, dst_at):
            rdmas = []
            for k in range(1, N_DEV):
                t = (my + k) % N_DEV
                rdma = pltpu.make_async_remote_copy(
                    src_ref=src_at(k, t),
                    dst_ref=dst_at(k, t),
                    send_sem=send_sems.at[round_idx, k - 1],
                    recv_sem=recv_sems.at[round_idx, k - 1],
                    device_id=(t,),
                    device_id_type=pl.DeviceIdType.MESH,
                )
                rdma.start()
                rdmas.append(rdma)
            for rdma in rdmas:
                rdma.wait()

        def allgather(round_idx, Xl):
            exchange(
                round_idx,
                lambda k, t: Xl.at[pl.ds(my * M, M), :],
                lambda k, t: Xl.at[pl.ds(my * M, M), :],
            )

        def reduce_scatter(round_idx, r):
            for k in range(1, N_DEV):
                t = (my + k) % N_DEV
                sbufs[r, k - 1, :, :] = Pbuf[pl.ds(t * M, M), :].astype(
                    jnp.bfloat16
                )
            exchange(
                round_idx,
                lambda k, t: sbufs.at[r, k - 1],
                lambda k, t: rbufs.at[r, k - 1],
            )
            total = Pbuf[pl.ds(my * M, M), :]
            for k in range(N_DEV - 1):
                total = total + rbufs[r, k].astype(jnp.float32)
            return total

        def layer(Xl, win_ref, wout_ref):
            Xv = Xl[:, :]
            w_in = win_ref[:, :].astype(jnp.bfloat16)
            h = jnp.dot(Xv, w_in, preferred_element_type=jnp.float32)
            h = jnp.maximum(h, 0.0).astype(jnp.bfloat16)
            w_out = wout_ref[:, :].astype(jnp.bfloat16)
            Pbuf[:, :] = jnp.dot(h, w_out, preferred_element_type=jnp.float32)

        X0[pl.ds(my * M, M), :] = x_ref[:, :].astype(jnp.bfloat16)
        allgather(0, X0)

        layer(X0, win0_ref, wout0_ref)
        red = reduce_scatter(1, 0)
        X1[pl.ds(my * M, M), :] = red.astype(jnp.bfloat16)
        allgather(2, X1)

        layer(X1, win1_ref, wout1_ref)
        red = reduce_scatter(3, 1)
        X2[pl.ds(my * M, M), :] = red.astype(jnp.bfloat16)
        allgather(4, X2)

        layer(X2, win2_ref, wout2_ref)
        out_ref[:, :] = reduce_scatter(5, 2)

    return pl.pallas_call(
        body,
        out_shape=jax.ShapeDtypeStruct((M, D), jnp.float32),
        in_specs=[pl.BlockSpec(memory_space=pltpu.VMEM)] * 7,
        out_specs=pl.BlockSpec(memory_space=pltpu.VMEM),
        scratch_shapes=[
            pltpu.VMEM((B, D), jnp.bfloat16),
            pltpu.VMEM((B, D), jnp.bfloat16),
            pltpu.VMEM((B, D), jnp.bfloat16),
            pltpu.VMEM((B, D), jnp.float32),
            pltpu.VMEM((3, N_DEV - 1, M, D), jnp.bfloat16),
            pltpu.VMEM((3, N_DEV - 1, M, D), jnp.bfloat16),
            pltpu.SemaphoreType.DMA((6, N_DEV - 1)),
            pltpu.SemaphoreType.DMA((6, N_DEV - 1)),
        ],
        compiler_params=pltpu.CompilerParams(collective_id=0),
    )(x, Win0, Wout0, Win1, Wout1, Win2, Wout2)


# device time: 39686 ns/iter; 1.0710x vs baseline; 1.0710x over previous
import jax
import jax.numpy as jnp
from jax import lax
from jax.experimental import pallas as pl
from jax.experimental.pallas import tpu as pltpu

N_DEV = 4


def kernel(x, Win0, Wout0, Win1, Wout1, Win2, Wout2):
    M, D = x.shape
    B = N_DEV * M

    def body(x_ref, win0_ref, wout0_ref, win1_ref, wout1_ref, win2_ref,
             wout2_ref, out_ref, X0, X1, X2, prbuf, sbuf,
             send_sems, recv_sems):
        my = lax.axis_index("i")

        barrier_sem = pltpu.get_barrier_semaphore()
        for k in range(1, N_DEV):
            pl.semaphore_signal(
                barrier_sem, inc=1,
                device_id=((my + k) % N_DEV,),
                device_id_type=pl.DeviceIdType.MESH,
            )
        pl.semaphore_wait(barrier_sem, N_DEV - 1)

        def compute_chunk(xc, win_ref, wout_ref):
            w_in = win_ref[:, :].astype(jnp.bfloat16)
            h = jnp.dot(xc, w_in, preferred_element_type=jnp.float32)
            h = jnp.maximum(h, 0.0).astype(jnp.bfloat16)
            w_out = wout_ref[:, :].astype(jnp.bfloat16)
            return jnp.dot(h, w_out, preferred_element_type=jnp.float32)

        def layer_step(l, Xl, win_ref, wout_ref, x_own):
            Xl[pl.ds(my * M, M), :] = x_own
            xdescs = []
            for k in range(1, N_DEV):
                t = (my + k) % N_DEV
                de = pltpu.make_async_remote_copy(
                    src_ref=Xl.at[pl.ds(my * M, M), :],
                    dst_ref=Xl.at[pl.ds(my * M, M), :],
                    send_sem=send_sems.at[2 * l, k - 1],
                    recv_sem=recv_sems.at[2 * l, k - 1],
                    device_id=(t,),
                    device_id_type=pl.DeviceIdType.MESH,
                )
                de.start()
                xdescs.append(de)

            total = compute_chunk(x_own, win_ref, wout_ref)

            pdescs = []
            for k in (1, 3, 2):
                xdescs[k - 1].wait_recv()
                s = (my - k) % N_DEV
                xk = Xl[pl.ds(s * M, M), :]
                pk = compute_chunk(xk, win_ref, wout_ref)
                slot = 3 - k
                sbuf[l, slot, :, :] = pk.astype(jnp.bfloat16)
                de = pltpu.make_async_remote_copy(
                    src_ref=sbuf.at[l, slot],
                    dst_ref=prbuf.at[l, slot],
                    send_sem=send_sems.at[2 * l + 1, slot],
                    recv_sem=recv_sems.at[2 * l + 1, slot],
                    device_id=(s,),
                    device_id_type=pl.DeviceIdType.MESH,
                )
                de.start()
                pdescs.append(de)

            for de, k in zip(pdescs, (1, 3, 2)):
                de.wait_recv()
                total = total + prbuf[l, 3 - k].astype(jnp.float32)
            for de in xdescs + pdescs:
                de.wait_send()
            return total

        x0 = x_ref[:, :].astype(jnp.bfloat16)
        r0 = layer_step(0, X0, win0_ref, wout0_ref, x0)
        r1 = layer_step(1, X1, win1_ref, wout1_ref, r0.astype(jnp.bfloat16))
        r2 = layer_step(2, X2, win2_ref, wout2_ref, r1.astype(jnp.bfloat16))
        out_ref[:, :] = r2

    return pl.pallas_call(
        body,
        out_shape=jax.ShapeDtypeStruct((M, D), jnp.float32),
        in_specs=[pl.BlockSpec(memory_space=pltpu.VMEM)] * 7,
        out_specs=pl.BlockSpec(memory_space=pltpu.VMEM),
        scratch_shapes=[
            pltpu.VMEM((B, D), jnp.bfloat16),
            pltpu.VMEM((B, D), jnp.bfloat16),
            pltpu.VMEM((B, D), jnp.bfloat16),
            pltpu.VMEM((3, N_DEV - 1, M, D), jnp.bfloat16),
            pltpu.VMEM((3, N_DEV - 1, M, D), jnp.bfloat16),
            pltpu.SemaphoreType.DMA((6, N_DEV - 1)),
            pltpu.SemaphoreType.DMA((6, N_DEV - 1)),
        ],
        compiler_params=pltpu.CompilerParams(collective_id=0),
    )(x, Win0, Wout0, Win1, Wout1, Win2, Wout2)
